# baseline (speedup 1.0000x reference)
import numpy as np
import jax
import jax.numpy as jnp
from jax.experimental import pallas as pl
from jax.experimental.pallas import tpu as pltpu
from jax.sharding import Mesh, NamedSharding, PartitionSpec as P

_TB = 512


def _lrelu(x):
    return jnp.where(x >= 0, x, 0.01 * x)


def _mm_act_body(x_ref, w_ref, b_ref, o_ref):
    y = jnp.dot(x_ref[...], w_ref[...], preferred_element_type=jnp.float32)
    o_ref[...] = _lrelu(y + b_ref[...])


def _mm_body(x_ref, w_ref, b_ref, o_ref):
    y = jnp.dot(x_ref[...], w_ref[...], preferred_element_type=jnp.float32)
    o_ref[...] = y + b_ref[...]


def _mm_act_s_body(x_ref, s_ref, ws_ref, w_ref, b_ref, o_ref):
    y = jnp.dot(x_ref[...], w_ref[...], preferred_element_type=jnp.float32)
    o_ref[...] = _lrelu(y + s_ref[...] * ws_ref[...] + b_ref[...])


def _mm(body, x, w_rows, extras, w, b):
    bsz, k = x.shape
    n = w.shape[1]
    extra_specs = [pl.BlockSpec((_TB, 1), lambda i: (i, 0))
                   for _ in extras[:1]]
    extra_specs += [pl.BlockSpec((1, n), lambda i: (0, 0))
                    for _ in extras[1:]]
    return pl.pallas_call(
        body,
        grid=(bsz // _TB,),
        in_specs=[pl.BlockSpec((_TB, k), lambda i: (i, 0))]
        + extra_specs
        + [
            pl.BlockSpec((w_rows, n), lambda i: (0, 0)),
            pl.BlockSpec((1, n), lambda i: (0, 0)),
        ],
        out_specs=pl.BlockSpec((_TB, n), lambda i: (i, 0)),
        out_shape=jax.ShapeDtypeStruct((bsz, n), jnp.float32),
        compiler_params=pltpu.CompilerParams(
            dimension_semantics=("arbitrary",)),
    )(x, *extras, w, b)


def _pipeline(data, s, mu_W1, mu_b1, mu_W2, mu_b2, lv_W1, lv_b1, lv_W2,
              lv_b2, dec_W1, dec_w1s, dec_b1, dec_W2, dec_b2):
    k = data.shape[1]
    h = mu_W2.shape[0]
    h1mu = _mm(_mm_act_body, data, k, (), mu_W1, mu_b1)
    mu_h = _mm(_mm_body, h1mu, h, (), mu_W2, mu_b2)
    h1lv = _mm(_mm_act_body, data, k, (), lv_W1, lv_b1)
    logvar_h = _mm(_mm_body, h1lv, h, (), lv_W2, lv_b2)
    hdec = _mm(_mm_act_s_body, mu_h, h, (s, dec_w1s), dec_W1, dec_b1)
    data_reconst = _mm(_mm_body, hdec, h, (), dec_W2, dec_b2)
    return (data_reconst, mu_h, logvar_h)


def kernel(data, s, mu_W1, mu_b1, mu_W2, mu_b2, lv_W1, lv_b1, lv_W2, lv_b2,
           dec_W1, dec_b1, dec_W2, dec_b2):
    mesh = Mesh(np.asarray(jax.devices()[:2]), ("b",))
    rep = P()
    row = P("b", None)
    weights = (mu_W1, mu_b1.reshape(1, -1), mu_W2, mu_b2.reshape(1, -1),
               lv_W1, lv_b1.reshape(1, -1), lv_W2, lv_b2.reshape(1, -1),
               dec_W1, dec_W1[-1:], dec_b1.reshape(1, -1), dec_W2,
               dec_b2.reshape(1, -1))
    fn = jax.shard_map(
        _pipeline, mesh=mesh,
        in_specs=(row, row) + (rep,) * len(weights),
        out_specs=(row, row, row),
        check_vma=False,
    )
    return fn(data, s, *weights)


# speedup vs baseline: 1.4037x; 1.4037x over previous
import numpy as np
import jax
import jax.numpy as jnp
from jax.experimental import pallas as pl
from jax.experimental.pallas import tpu as pltpu
from jax.sharding import Mesh, NamedSharding, PartitionSpec as P

_TB = 512


def _lrelu(x):
    return jnp.where(x >= 0, x, 0.01 * x)


def _mm_act_body(x_ref, w_ref, b_ref, o_ref):
    y = jnp.dot(x_ref[...], w_ref[...], preferred_element_type=jnp.float32)
    o_ref[...] = _lrelu(y + b_ref[...])


def _mm_body(x_ref, w_ref, b_ref, o_ref):
    y = jnp.dot(x_ref[...], w_ref[...], preferred_element_type=jnp.float32)
    o_ref[...] = y + b_ref[...]


def _mm_act_s_body(x_ref, s_ref, ws_ref, w_ref, b_ref, o_ref):
    y = jnp.dot(x_ref[...], w_ref[...], preferred_element_type=jnp.float32)
    o_ref[...] = _lrelu(y + s_ref[...] * ws_ref[...] + b_ref[...])


def _mm(body, x, w_rows, extras, w, b):
    bsz, k = x.shape
    n = w.shape[1]
    extra_specs = [pl.BlockSpec((_TB, 1), lambda i: (i, 0))
                   for _ in extras[:1]]
    extra_specs += [pl.BlockSpec((1, n), lambda i: (0, 0))
                    for _ in extras[1:]]
    return pl.pallas_call(
        body,
        grid=(bsz // _TB,),
        in_specs=[pl.BlockSpec((_TB, k), lambda i: (i, 0))]
        + extra_specs
        + [
            pl.BlockSpec((w_rows, n), lambda i: (0, 0)),
            pl.BlockSpec((1, n), lambda i: (0, 0)),
        ],
        out_specs=pl.BlockSpec((_TB, n), lambda i: (i, 0)),
        out_shape=jax.ShapeDtypeStruct((bsz, n), jnp.float32),
        compiler_params=pltpu.CompilerParams(
            dimension_semantics=("arbitrary",)),
    )(x, *extras, w, b)


def _pipeline(data, s, mu_W1, mu_b1, mu_W2, mu_b2, lv_W1, lv_b1, lv_W2,
              lv_b2, dec_W1, dec_w1s, dec_b1, dec_W2, dec_b2):
    k = data.shape[1]
    h = mu_W2.shape[0]
    h1mu = _mm(_mm_act_body, data, k, (), mu_W1, mu_b1)
    mu_h = _mm(_mm_body, h1mu, h, (), mu_W2, mu_b2)
    h1lv = _mm(_mm_act_body, data, k, (), lv_W1, lv_b1)
    logvar_h = _mm(_mm_body, h1lv, h, (), lv_W2, lv_b2)
    hdec = _mm(_mm_act_s_body, mu_h, h, (s, dec_w1s), dec_W1, dec_b1)
    data_reconst = _mm(_mm_body, hdec, h, (), dec_W2, dec_b2)
    return (data_reconst, mu_h, logvar_h)


def kernel(data, s, mu_W1, mu_b1, mu_W2, mu_b2, lv_W1, lv_b1, lv_W2, lv_b2,
           dec_W1, dec_b1, dec_W2, dec_b2):
    return _pipeline(data, s, mu_W1, mu_b1.reshape(1, -1), mu_W2,
                     mu_b2.reshape(1, -1), lv_W1, lv_b1.reshape(1, -1),
                     lv_W2, lv_b2.reshape(1, -1), dec_W1, dec_W1[-1:],
                     dec_b1.reshape(1, -1), dec_W2, dec_b2.reshape(1, -1))


# speedup vs baseline: 1.4104x; 1.0047x over previous
import jax
import jax.numpy as jnp
from jax.experimental import pallas as pl
from jax.experimental.pallas import tpu as pltpu


def _lrelu(x):
    return jnp.where(x >= 0, x, 0.01 * x)


def _mm_act_body(x_ref, w_ref, b_ref, o_ref):
    y = jnp.dot(x_ref[...], w_ref[...], preferred_element_type=jnp.float32)
    o_ref[...] = _lrelu(y + b_ref[...])


def _mm_body(x_ref, w_ref, b_ref, o_ref):
    y = jnp.dot(x_ref[...], w_ref[...], preferred_element_type=jnp.float32)
    o_ref[...] = y + b_ref[...]


def _mu2_dec1_body(x_ref, s_ref, w2_ref, b2_ref, wd_ref, wds_ref, bd_ref,
                   mu_ref, hd_ref):
    mu = jnp.dot(x_ref[...], w2_ref[...], preferred_element_type=jnp.float32)
    mu = mu + b2_ref[...]
    mu_ref[...] = mu
    y = jnp.dot(mu, wd_ref[...], preferred_element_type=jnp.float32)
    hd_ref[...] = _lrelu(y + s_ref[...] * wds_ref[...] + bd_ref[...])


def _mm(body, x, w, b, tb):
    bsz, k = x.shape
    n = w.shape[1]
    return pl.pallas_call(
        body,
        grid=(bsz // tb,),
        in_specs=[
            pl.BlockSpec((tb, k), lambda i: (i, 0)),
            pl.BlockSpec((k, n), lambda i: (0, 0)),
            pl.BlockSpec((1, n), lambda i: (0, 0)),
        ],
        out_specs=pl.BlockSpec((tb, n), lambda i: (i, 0)),
        out_shape=jax.ShapeDtypeStruct((bsz, n), jnp.float32),
        compiler_params=pltpu.CompilerParams(
            dimension_semantics=("arbitrary",)),
    )(x, w, b)


def _mu2_dec1(h1mu, s, mu_W2, mu_b2, dec_W1, dec_w1s, dec_b1, tb):
    bsz, k = h1mu.shape
    n = mu_W2.shape[1]
    outs = pl.pallas_call(
        _mu2_dec1_body,
        grid=(bsz // tb,),
        in_specs=[
            pl.BlockSpec((tb, k), lambda i: (i, 0)),
            pl.BlockSpec((tb, 1), lambda i: (i, 0)),
            pl.BlockSpec((k, n), lambda i: (0, 0)),
            pl.BlockSpec((1, n), lambda i: (0, 0)),
            pl.BlockSpec((n, n), lambda i: (0, 0)),
            pl.BlockSpec((1, n), lambda i: (0, 0)),
            pl.BlockSpec((1, n), lambda i: (0, 0)),
        ],
        out_specs=[
            pl.BlockSpec((tb, n), lambda i: (i, 0)),
            pl.BlockSpec((tb, n), lambda i: (i, 0)),
        ],
        out_shape=[
            jax.ShapeDtypeStruct((bsz, n), jnp.float32),
            jax.ShapeDtypeStruct((bsz, n), jnp.float32),
        ],
        compiler_params=pltpu.CompilerParams(
            dimension_semantics=("arbitrary",)),
    )(h1mu, s, mu_W2, mu_b2, dec_W1, dec_w1s, dec_b1)
    return outs


def kernel(data, s, mu_W1, mu_b1, mu_W2, mu_b2, lv_W1, lv_b1, lv_W2, lv_b2,
           dec_W1, dec_b1, dec_W2, dec_b2):
    h1mu = _mm(_mm_act_body, data, mu_W1, mu_b1.reshape(1, -1), 512)
    h1lv = _mm(_mm_act_body, data, lv_W1, lv_b1.reshape(1, -1), 512)
    mu_h, hdec = _mu2_dec1(h1mu, s, mu_W2, mu_b2.reshape(1, -1), dec_W1,
                           dec_W1[-1:], dec_b1.reshape(1, -1), 256)
    logvar_h = _mm(_mm_body, h1lv, lv_W2, lv_b2.reshape(1, -1), 1024)
    data_reconst = _mm(_mm_body, hdec, dec_W2, dec_b2.reshape(1, -1), 512)
    return (data_reconst, mu_h, logvar_h)
